# baseline (speedup 1.0000x reference)
import functools

import jax
import jax.numpy as jnp
from jax import lax
from jax.experimental import pallas as pl
from jax.experimental.pallas import tpu as pltpu
from jax.experimental.pallas import tpu_sc as plsc

R = 16384
J = 50
D = 64
NUM_WORKERS = 32
RB = 128
N_RB = R // RB
RB_PER_W = N_RB // NUM_WORKERS
V = 1000000


def _build_transpose():
    N_FULL = 7808
    PER_W = N_FULL // NUM_WORKERS
    mesh = plsc.VectorSubcoreMesh(core_axis_name="c", subcore_axis_name="s")

    @functools.partial(
        pl.kernel,
        mesh=mesh,
        out_type=jax.ShapeDtypeStruct((V // 2, 128), jnp.float32),
        scratch_types=[
            pltpu.VMEM((4, D, 128), jnp.float32),
            pltpu.VMEM((4, D, 128), jnp.float32),
            pltpu.VMEM((D, D), jnp.float32),
            pltpu.SemaphoreType.DMA,
            pltpu.SemaphoreType.DMA,
        ],
        compiler_params=pltpu.CompilerParams(needs_layout_passes=False),
    )
    def transpose_kernel(tab_t_hbm, tail_t_hbm, out_hbm, blk_v, t_v, tail_v,
                         gsem, wsem):
        wid = lax.axis_index("s") * 2 + lax.axis_index("c")
        iota16 = lax.iota(jnp.int32, 16)
        u0 = wid * PER_W

        def start_read(u, b):
            pltpu.async_copy(
                tab_t_hbm.at[:, pl.ds(u * 128, 128)], blk_v.at[b], gsem
            )

        def wait_read(b):
            pltpu.make_async_copy(
                tab_t_hbm.at[:, pl.ds(0, 128)], blk_v.at[b], gsem
            ).wait()

        def start_write(u, b):
            pltpu.async_copy(
                t_v.at[b], out_hbm.at[pl.ds(u * 64, 64)], wsem
            )

        def wait_write(b):
            pltpu.make_async_copy(
                t_v.at[b], out_hbm.at[pl.ds(0, 64)], wsem
            ).wait()

        def transpose(src, b, n_r):
            for cg in range(8):
                dvec = (cg % 4) * 16 + iota16
                delta = 1 if cg >= 4 else 0
                sl = pl.ds(cg * 16, 16)

                @plsc.parallel_loop(0, n_r, unroll=8)
                def _(r):
                    vsplat = jnp.full((16,), 0, jnp.int32) + (2 * r + delta)
                    v = plsc.load_gather(src, [dvec, vsplat])
                    t_v[b, r, sl] = v

        for t in range(3):
            start_read(u0 + t, t)
        for t in range(4):
            wait_read(t)
            start_read(u0 + t + 3, (t + 3) % 4)
            transpose(blk_v.at[t], t, D)
            start_write(u0 + t, t)

        def body(i, carry):
            for u_ in range(4):
                t = 4 * i + u_
                wait_read(u_)

                @pl.when(t + 3 <= PER_W - 1)
                def _():
                    start_read(u0 + t + 3, (u_ + 3) % 4)

                wait_write(u_)
                transpose(blk_v.at[u_], u_, D)
                start_write(u0 + t, u_)
            return carry
        lax.fori_loop(1, PER_W // 4, body, 0)
        for b in range(4):
            wait_write(b)

        @pl.when(wid < 4)
        def _():
            u = N_FULL + wid
            start_read(u, 0)
            wait_read(0)
            transpose(blk_v.at[0], 0, D)
            start_write(u, 0)
            wait_write(0)

        @pl.when(wid == 4)
        def _():
            u = N_FULL + 4
            pltpu.sync_copy(tail_t_hbm, tail_v)
            transpose(tail_v, 0, 32)
            pltpu.async_copy(
                t_v.at[0, pl.ds(0, 32)], out_hbm.at[pl.ds(u * 64, 32)], wsem
            )
            pltpu.make_async_copy(
                t_v.at[0, pl.ds(0, 32)], out_hbm.at[pl.ds(0, 32)], wsem
            ).wait()

    return transpose_kernel


def _build_gather():
    mesh = plsc.VectorSubcoreMesh(core_axis_name="c", subcore_axis_name="s")

    @functools.partial(
        pl.kernel,
        mesh=mesh,
        out_type=jax.ShapeDtypeStruct((J, D, R), jnp.float32),
        scratch_types=[
            pltpu.VMEM((RB * J,), jnp.int32),
            pltpu.VMEM((J, RB), jnp.int32),
            pltpu.VMEM((J, RB), jnp.int32),
            pltpu.VMEM((4, RB, 128), jnp.float32),
            pltpu.VMEM((4, D, RB), jnp.float32),
            pltpu.SemaphoreType.DMA,
            pltpu.SemaphoreType.DMA,
        ],
        compiler_params=pltpu.CompilerParams(needs_layout_passes=False),
    )
    def gather_kernel(idx_hbm, table2_hbm, out_hbm,
                      slab_v, idxcol_v, off_v, rows_v, t_v, gsem, wsem):
        wid = lax.axis_index("s") * 2 + lax.axis_index("c")
        iota16 = lax.iota(jnp.int32, 16)

        def start_gather(j, b):
            pltpu.async_copy(table2_hbm.at[idxcol_v.at[j]], rows_v.at[b], gsem)

        def wait_gather(b):
            pltpu.make_async_copy(
                table2_hbm.at[idxcol_v.at[0]], rows_v.at[b], gsem
            ).wait()

        def start_write(j, rb, b):
            pltpu.async_copy(
                t_v.at[b], out_hbm.at[j, :, pl.ds(rb * RB, RB)], wsem
            )

        def wait_write(b):
            pltpu.make_async_copy(
                t_v.at[b], out_hbm.at[0, :, pl.ds(0, RB)], wsem
            ).wait()

        def transpose(j, b):
            for kg in range(8):
                kvec = kg * 16 + iota16
                offv = off_v[j, pl.ds(kg * 16, 16)]
                sl = pl.ds(kg * 16, 16)

                @plsc.parallel_loop(0, D, unroll=8)
                def _(d):
                    v = plsc.load_gather(rows_v.at[b], [kvec, offv + d])
                    t_v[b, d, sl] = v

        def per_rb(i, carry):
            rb = wid * RB_PER_W + i
            pltpu.sync_copy(idx_hbm.at[pl.ds(rb * RB * J, RB * J)], slab_v)

            def extract_j(j, c):
                for kg in range(8):
                    av = (kg * 16 + iota16) * J + j
                    v = plsc.load_gather(slab_v, [av])
                    idxcol_v[j, pl.ds(kg * 16, 16)] = v >> 1
                    off_v[j, pl.ds(kg * 16, 16)] = (v & 1) << 6
                return c
            lax.fori_loop(0, J, extract_j, 0)

            for j in range(3):
                start_gather(j, j)
            for j in range(4):
                wait_gather(j)
                start_gather(j + 3, (j + 3) % 4)
                transpose(j, j)
                start_write(j, rb, j)

            def body(i2, c):
                for u in range(4):
                    j = 4 * i2 + u
                    wait_gather(u)

                    @pl.when(j + 3 <= J - 1)
                    def _():
                        start_gather(j + 3, (u + 3) % 4)

                    wait_write(u)
                    transpose(j, u)
                    start_write(j, rb, u)
                return c
            lax.fori_loop(1, 12, body, 0)

            for j in (48, 49):
                b = j % 4
                wait_gather(b)
                wait_write(b)
                transpose(j, b)
                start_write(j, rb, b)
            for b in range(4):
                wait_write(b)
            return carry

        lax.fori_loop(0, RB_PER_W, per_rb, 0)

    return gather_kernel


def kernel(x, table):
    idx = x.reshape(R * J).astype(jnp.int32)
    table2 = _build_transpose()(table.T, table[V - D:].T)
    out = _build_gather()(idx, table2)
    return out.transpose(2, 0, 1)


# speedup vs baseline: 1.2189x; 1.2189x over previous
import functools

import jax
import jax.numpy as jnp
from jax import lax
from jax.experimental import pallas as pl
from jax.experimental.pallas import tpu as pltpu
from jax.experimental.pallas import tpu_sc as plsc

EMBED_D = 64
NUM_WORKERS = 32
CHUNK = 128
NBUF = 8
LAG = 4


def _build_gather(b_total: int, d: int):
    b_per_w = b_total // NUM_WORKERS
    n_chunks = b_per_w // CHUNK
    n_main = n_chunks - 2 * LAG
    n_outer = n_main // NBUF
    assert n_main % NBUF == 0
    mesh = plsc.VectorSubcoreMesh(core_axis_name="c", subcore_axis_name="s")

    @functools.partial(
        pl.kernel,
        mesh=mesh,
        out_type=jax.ShapeDtypeStruct((b_total, d), jnp.float32),
        scratch_types=[
            pltpu.VMEM((b_per_w,), jnp.int32),
            pltpu.VMEM((NBUF, CHUNK, d), jnp.float32),
            pltpu.SemaphoreType.DMA,
            pltpu.SemaphoreType.DMA,
        ],
        compiler_params=pltpu.CompilerParams(use_tc_tiling_on_sc=False),
    )
    def gather_kernel(idx_hbm, table_hbm, out_hbm, idx_v, rows_v, gsem, wsem):
        wid = lax.axis_index("s") * 2 + lax.axis_index("c")
        base = wid * b_per_w
        pltpu.sync_copy(idx_hbm.at[pl.ds(base, b_per_w)], idx_v)

        def start_gather(j, b):
            pltpu.async_copy(
                table_hbm.at[idx_v.at[pl.ds(j * CHUNK, CHUNK)]], rows_v.at[b], gsem
            )

        def wait_gather(b):
            pltpu.make_async_copy(
                table_hbm.at[idx_v.at[pl.ds(0, CHUNK)]], rows_v.at[b], gsem
            ).wait()

        def start_write(j, b):
            pltpu.async_copy(
                rows_v.at[b], out_hbm.at[pl.ds(base + j * CHUNK, CHUNK)], wsem
            )

        def wait_write(b):
            pltpu.make_async_copy(
                rows_v.at[b], out_hbm.at[pl.ds(base, CHUNK)], wsem
            ).wait()

        for s in range(LAG):
            start_gather(s, s)

        for s in range(LAG):
            wait_gather(s % NBUF)
            start_write(s, s % NBUF)
            start_gather(s + LAG, (s + LAG) % NBUF)

        def body(i, carry):
            s0 = LAG + i * NBUF
            for u in range(NBUF):
                b = (LAG + u) % NBUF
                s = s0 + u
                wait_gather(b)
                start_write(s, b)
                wait_write((LAG + u) % NBUF)
                start_gather(s + LAG, u % NBUF)
            return carry

        lax.fori_loop(0, n_outer, body, 0)

        for s in range(n_chunks - LAG, n_chunks):
            b = s % NBUF
            wait_gather(b)
            start_write(s, b)
            wait_write(b)

        for s in range(LAG):
            wait_write(s)

    return gather_kernel


def kernel(x, table):
    b_total = x.shape[0] * x.shape[1]
    d = table.shape[1]
    idx = x.reshape(b_total).astype(jnp.int32)
    out = _build_gather(b_total, d)(idx, table)
    return out.reshape(x.shape + (d,))
